# baseline (speedup 1.0000x reference)
import functools

import jax
import jax.numpy as jnp
from jax import lax
from jax.experimental import pallas as pl
from jax.experimental.pallas import tpu as pltpu
from jax.experimental.pallas import tpu_sc as plsc

PAD = 0.0

_B, _L = 16, 4096
_N = _B * _L
_NW = 32
_CHUNK = _N // _NW
_LANES = 16


def _sc_body(x_hbm, out_hbm, mask_hbm, x_v, m_v):
    wid = lax.axis_index("s") * 2 + lax.axis_index("c")
    base = wid * _CHUNK
    pltpu.sync_copy(x_hbm.at[pl.ds(base, _CHUNK)], x_v)
    absmask = jnp.full((_LANES,), 0x7FFFFFFF, dtype=jnp.int32)
    one = jnp.full((_LANES,), 1, dtype=jnp.int32)
    for i in range(_CHUNK // _LANES):
        iv = x_v[pl.ds(i * _LANES, _LANES)]
        t = lax.min(iv & absmask, one)
        m_v[pl.ds(i * _LANES, _LANES)] = one - t
    pltpu.sync_copy(x_v, out_hbm.at[pl.ds(base, _CHUNK)])
    pltpu.sync_copy(m_v, mask_hbm.at[pl.ds(base, _CHUNK)])


def _sc_call(x_flat_i32):
    mesh = plsc.VectorSubcoreMesh(core_axis_name="c", subcore_axis_name="s")
    k = functools.partial(
        pl.kernel,
        mesh=mesh,
        out_type=(
            jax.ShapeDtypeStruct((_N,), jnp.int32),
            jax.ShapeDtypeStruct((_N,), jnp.int32),
        ),
        scratch_types=[
            pltpu.VMEM((_CHUNK,), jnp.int32),
            pltpu.VMEM((_CHUNK,), jnp.int32),
        ],
    )(_sc_body)
    return k(x_flat_i32)


def kernel(tokens_batch):
    B, L = tokens_batch.shape
    xi = lax.bitcast_convert_type(tokens_batch, jnp.int32).reshape(-1)
    out_i32, m32 = _sc_call(xi)
    out = lax.bitcast_convert_type(out_i32.reshape(B, L), jnp.float32)
    mask = m32.astype(jnp.bool_).reshape(B, L)
    return (out, mask)


# speedup vs baseline: 5.9450x; 5.9450x over previous
import jax
import jax.numpy as jnp
from jax.experimental import pallas as pl

PAD = 0.0


def _pad_mask_kernel(x_ref, out_ref, mask_ref):
    x = x_ref[...]
    out_ref[...] = x
    mask_ref[...] = x == PAD


def kernel(tokens_batch):
    B, L = tokens_batch.shape
    out, mask = pl.pallas_call(
        _pad_mask_kernel,
        out_shape=(
            jax.ShapeDtypeStruct((B, L), tokens_batch.dtype),
            jax.ShapeDtypeStruct((B, L), jnp.bool_),
        ),
    )(tokens_batch)
    return (out, mask)


# speedup vs baseline: 6.2436x; 1.0502x over previous
import jax
import jax.numpy as jnp
from jax.experimental import pallas as pl

PAD = 0.0


def _pad_mask_kernel(x_ref, out_ref, mask_ref):
    x = x_ref[...]
    out_ref[...] = x
    mask_ref[...] = (x == PAD).astype(jnp.int8)


def kernel(tokens_batch):
    B, L = tokens_batch.shape
    out, mask8 = pl.pallas_call(
        _pad_mask_kernel,
        out_shape=(
            jax.ShapeDtypeStruct((B, L), tokens_batch.dtype),
            jax.ShapeDtypeStruct((B, L), jnp.int8),
        ),
    )(tokens_batch)
    return (out, mask8.astype(jnp.bool_))
